# baseline (speedup 1.0000x reference)
import functools

import jax
import jax.numpy as jnp
from jax import lax
from jax.experimental import pallas as pl
from jax.experimental.pallas import tpu as pltpu
from jax.experimental.pallas import tpu_sc as plsc

NUM_USERS = 5000
NUM_ITEMS = 5000
N_NODES = NUM_USERS + NUM_ITEMS
N_EDGES = 320000
D = 128
N_LAYERS = 4
BATCH = 4096
BIAS_SCALE = 0.5
RES_ALPHA = 0.3
EPS = 1e-5
IN_DIM = 5 * D
HIDDEN = 128

NC = 2
NS = 16
NW = NC * NS
LANES = 16

EPW = N_EDGES // NW
EC = 64
NCHUNK = 161
EPW_PAD = NCHUNK * EC
PAD = EPW_PAD - EPW
RPT = 624
TAIL = N_NODES - RPT * NS
ZCH = RPT // EC
ZREM = RPT - ZCH * EC

B_PER_W = BATCH // NW

@functools.cache
def _get_mesh():
    return plsc.VectorSubcoreMesh(
        core_axis_name="c", subcore_axis_name="s",
        num_cores=NC, num_subcores=NS)


@functools.cache
def _propagate_kernel():
    return pl.kernel(
        _propagate_body,
        out_type=jax.ShapeDtypeStruct((NC, N_NODES, D), jnp.float32),
        mesh=_get_mesh(),
        scratch_types=[
            pltpu.VMEM((5, 2, EC), jnp.int32),
            pltpu.VMEM((5, EC), jnp.float32),
            pltpu.VMEM((5, EC), jnp.int32),
            pltpu.VMEM((EC, D), jnp.float32),
            pltpu.VMEM((EC, D), jnp.float32),
            pltpu.VMEM((EC, D), jnp.float32),
            pltpu.VMEM((EC, D), jnp.float32),
            pltpu.VMEM((EC, D), jnp.float32),
            pltpu.VMEM_SHARED((N_NODES, D), jnp.float32),
            [pltpu.SemaphoreType.DMA] * 5,
            [pltpu.SemaphoreType.DMA] * 5,
            [pltpu.SemaphoreType.DMA] * 5,
        ],
    )


def _propagate_body(x_hbm, e_hbm, ew_hbm, out_hbm,
                    ebuf_v, ewbuf_v, dstc_v, rows0_v, rows1_v, rows2_v,
                    rows3_v, rows4_v, acc_sh, gsem, esem, ssem):
    c = lax.axis_index("c")
    s = lax.axis_index("s")
    wid = c * NS + s

    zero16 = jnp.zeros((LANES,), jnp.float32)
    for r in range(EC):
        for j in range(D // LANES):
            rows0_v[r, pl.ds(j * LANES, LANES)] = zero16
    for k in range(ZCH):
        acc_off = s * RPT + k * EC
        pltpu.sync_copy(rows0_v, acc_sh.at[pl.ds(acc_off, EC)])
    pltpu.sync_copy(rows0_v.at[pl.ds(0, ZREM)],
                    acc_sh.at[pl.ds(s * RPT + ZCH * EC, ZREM)])

    @pl.when(s == NS - 1)
    def _zero_tail():
        pltpu.sync_copy(rows0_v.at[pl.ds(0, TAIL)],
                        acc_sh.at[pl.ds(RPT * NS, TAIL)])

    plsc.subcore_barrier()

    rows = (rows0_v, rows1_v, rows2_v, rows3_v, rows4_v)

    def scale(p):
        buf = rows[p]
        for j in range(EC // LANES):
            sl = pl.ds(j * LANES, LANES)
            dstc_v[p, sl] = ebuf_v[p, 1, sl]
        for g in range(EC // LANES):
            wvec = ewbuf_v[p, pl.ds(g * LANES, LANES)]
            for l in range(LANES):
                i = g * LANES + l
                wi = wvec[l]
                for j in range(D // LANES):
                    sl = pl.ds(j * LANES, LANES)
                    buf[i, sl] = buf[i, sl] * wi

    def issue_scatter(p):
        pltpu.async_copy(rows[p], acc_sh.at[dstc_v.at[p]], ssem[p],
                         add=True)

    def wait_scatter(p):
        pltpu.make_async_copy(rows[p], acc_sh.at[dstc_v.at[p]],
                              ssem[p]).wait()

    def issue_edges(k, p):
        pltpu.async_copy(e_hbm.at[wid, k], ebuf_v.at[p], esem[p])
        pltpu.async_copy(ew_hbm.at[wid, k], ewbuf_v.at[p], esem[p])

    def wait_edges(p):
        pltpu.make_async_copy(e_hbm.at[wid, 0], ebuf_v.at[p],
                              esem[p]).wait()
        pltpu.make_async_copy(ew_hbm.at[wid, 0], ewbuf_v.at[p],
                              esem[p]).wait()

    def issue_gather(p):
        pltpu.async_copy(x_hbm.at[ebuf_v.at[p, 0]], rows[p], gsem[p])

    def wait_gather(p):
        pltpu.make_async_copy(x_hbm.at[ebuf_v.at[p, 0]], rows[p],
                              gsem[p]).wait()

    pltpu.sync_copy(e_hbm.at[wid, 0], ebuf_v.at[0])
    pltpu.sync_copy(ew_hbm.at[wid, 0], ewbuf_v.at[0])
    issue_gather(0)
    issue_edges(1, 1)
    issue_edges(2, 2)
    issue_edges(3, 3)

    issue_edges(4, 4)
    wait_edges(1)
    issue_gather(1)
    wait_edges(2)
    issue_gather(2)
    wait_edges(3)
    issue_gather(3)
    wait_gather(0)
    scale(0)
    issue_scatter(0)

    issue_edges(5, 0)
    wait_edges(4)
    issue_gather(4)
    wait_gather(1)
    scale(1)
    issue_scatter(1)

    def steady(k, p):
        p3 = (p + 3) % 5
        issue_edges(k + 4, (p + 4) % 5)
        wait_scatter(p3)
        wait_edges(p3)
        issue_gather(p3)
        wait_gather(p)
        scale(p)
        issue_scatter(p)

    def body(t, carry):
        k = 5 * t + 2
        steady(k, 2)
        steady(k + 1, 3)
        steady(k + 2, 4)
        steady(k + 3, 0)
        steady(k + 4, 1)
        return carry

    lax.fori_loop(0, (NCHUNK - 6) // 5, body, 0)

    wait_scatter(0)
    wait_edges(0)
    issue_gather(0)
    wait_gather(2)
    scale(2)
    issue_scatter(2)

    wait_scatter(1)
    wait_gather(3)
    scale(3)
    issue_scatter(3)

    wait_scatter(2)
    wait_gather(4)
    scale(4)
    issue_scatter(4)

    wait_scatter(3)
    wait_gather(0)
    scale(0)
    issue_scatter(0)

    wait_scatter(4)
    wait_scatter(0)

    plsc.subcore_barrier()
    for k in range(ZCH):
        r0 = s * RPT + k * EC
        pltpu.sync_copy(acc_sh.at[pl.ds(r0, EC)], rows0_v)
        pltpu.sync_copy(rows0_v, out_hbm.at[c, pl.ds(r0, EC)])
    rrem = s * RPT + ZCH * EC
    pltpu.sync_copy(acc_sh.at[pl.ds(rrem, ZREM)], rows0_v.at[pl.ds(0, ZREM)])
    pltpu.sync_copy(rows0_v.at[pl.ds(0, ZREM)],
                    out_hbm.at[c, pl.ds(rrem, ZREM)])

    @pl.when(s == NS - 1)
    def _write_tail():
        pltpu.sync_copy(acc_sh.at[pl.ds(RPT * NS, TAIL)],
                        rows1_v.at[pl.ds(0, TAIL)])
        pltpu.sync_copy(rows1_v.at[pl.ds(0, TAIL)],
                        out_hbm.at[c, pl.ds(RPT * NS, TAIL)])


@functools.cache
def _batch_gather_kernel():
    return pl.kernel(
        _batch_gather_body,
        out_type=jax.ShapeDtypeStruct((4, BATCH, D), jnp.float32),
        mesh=_get_mesh(),
        scratch_types=[
            pltpu.VMEM((B_PER_W,), jnp.int32),
            pltpu.VMEM((B_PER_W,), jnp.int32),
            pltpu.VMEM((B_PER_W,), jnp.int32),
            pltpu.VMEM((B_PER_W, D), jnp.float32),
            pltpu.VMEM((B_PER_W, D), jnp.float32),
            pltpu.VMEM((B_PER_W, D), jnp.float32),
            pltpu.VMEM((B_PER_W, D), jnp.float32),
            pltpu.SemaphoreType.DMA,
        ],
    )


def _batch_gather_body(om_hbm, ubw_hbm, ibw_hbm, users_hbm, items_hbm,
                       out_hbm,
                       uidx_v, iidx_v, iidx2_v, bu_v, bi_v, bub_v, bib_v, sem):
    c = lax.axis_index("c")
    s = lax.axis_index("s")
    wid = c * NS + s
    base = pl.multiple_of(wid * B_PER_W, 8)

    pltpu.sync_copy(users_hbm.at[pl.ds(base, B_PER_W)], uidx_v)
    pltpu.sync_copy(items_hbm.at[pl.ds(base, B_PER_W)], iidx_v)
    off16 = jnp.full((LANES,), NUM_USERS, jnp.int32)
    for j in range(B_PER_W // LANES):
        sl = pl.ds(j * LANES, LANES)
        iidx2_v[sl] = iidx_v[sl] + off16

    cu = pltpu.async_copy(om_hbm.at[uidx_v], bu_v, sem)
    ci = pltpu.async_copy(om_hbm.at[iidx2_v], bi_v, sem)
    cub = pltpu.async_copy(ubw_hbm.at[uidx_v], bub_v, sem)
    cib = pltpu.async_copy(ibw_hbm.at[iidx_v], bib_v, sem)
    cu.wait()
    ci.wait()
    cub.wait()
    cib.wait()
    pltpu.sync_copy(bu_v, out_hbm.at[0, pl.ds(base, B_PER_W)])
    pltpu.sync_copy(bi_v, out_hbm.at[1, pl.ds(base, B_PER_W)])
    pltpu.sync_copy(bub_v, out_hbm.at[2, pl.ds(base, B_PER_W)])
    pltpu.sync_copy(bib_v, out_hbm.at[3, pl.ds(base, B_PER_W)])


_CROWS = 1000


def _combine_body(p_ref, s_ref, x_ref, sum_ref):
    x = p_ref[0] + p_ref[1]
    x_ref[...] = x
    sum_ref[...] = s_ref[...] + x


def _combine(p, ssum):
    return pl.pallas_call(
        _combine_body,
        grid=(N_NODES // _CROWS,),
        in_specs=[
            pl.BlockSpec((NC, _CROWS, D), lambda i: (0, i, 0)),
            pl.BlockSpec((_CROWS, D), lambda i: (i, 0)),
        ],
        out_specs=[
            pl.BlockSpec((_CROWS, D), lambda i: (i, 0)),
            pl.BlockSpec((_CROWS, D), lambda i: (i, 0)),
        ],
        out_shape=[
            jax.ShapeDtypeStruct((N_NODES, D), jnp.float32),
            jax.ShapeDtypeStruct((N_NODES, D), jnp.float32),
        ],
    )(p, ssum)


def _combine_last_body(p_ref, s_ref, out_ref):
    out_ref[...] = (s_ref[...] + p_ref[0] + p_ref[1]) * (1.0 / (N_LAYERS + 1))


def _combine_last(p, ssum):
    return pl.pallas_call(
        _combine_last_body,
        grid=(N_NODES // _CROWS,),
        in_specs=[
            pl.BlockSpec((NC, _CROWS, D), lambda i: (0, i, 0)),
            pl.BlockSpec((_CROWS, D), lambda i: (i, 0)),
        ],
        out_specs=pl.BlockSpec((_CROWS, D), lambda i: (i, 0)),
        out_shape=jax.ShapeDtypeStruct((N_NODES, D), jnp.float32),
    )(p, ssum)


def _mlp_body(u_ref, i_ref, ub_ref, ib_ref, om_ref, gam_ref, bet_ref,
              w1_ref, b1_ref, w2_ref, b2_ref, out_ref):
    u = u_ref[...]
    it = i_ref[...]
    ub = ub_ref[...] * BIAS_SCALE
    ib = ib_ref[...] * BIAS_SCALE
    om = om_ref[...]
    g_row = (jnp.sum(om[:NUM_USERS], axis=0, keepdims=True)
             + jnp.sum(om[NUM_USERS:], axis=0, keepdims=True)) * (1.0 / NUM_USERS)

    pieces = [u, it, ub, ib]
    tot = jnp.zeros((BATCH, 1), jnp.float32)
    for p in pieces:
        tot = tot + jnp.sum(p, axis=1, keepdims=True)
    tot = tot + jnp.sum(g_row)
    mu = tot * (1.0 / IN_DIM)

    ssq = jnp.zeros((BATCH, 1), jnp.float32)
    for p in pieces:
        d = p - mu
        ssq = ssq + jnp.sum(d * d, axis=1, keepdims=True)
    dg = g_row - mu
    ssq = ssq + jnp.sum(dg * dg, axis=1, keepdims=True)
    inv = lax.rsqrt(ssq * (1.0 / IN_DIM) + EPS)

    h = jnp.zeros((BATCH, HIDDEN), jnp.float32)
    for k in range(4):
        d = (pieces[k] - mu) * inv
        xn = d * gam_ref[:, k * D:(k + 1) * D] + bet_ref[:, k * D:(k + 1) * D]
        h = h + jnp.dot(xn, w1_ref[k * D:(k + 1) * D, :],
                        preferred_element_type=jnp.float32)
    dgn = dg * inv
    xng = dgn * gam_ref[:, 4 * D:] + bet_ref[:, 4 * D:]
    h = h + jnp.dot(xng, w1_ref[4 * D:, :], preferred_element_type=jnp.float32)
    h = jnp.maximum(h + b1_ref[...], 0.0)

    mlp = jnp.dot(h, w2_ref[...], preferred_element_type=jnp.float32) + b2_ref[...]
    dot = jnp.sum(u * it, axis=1, keepdims=True)
    out_ref[...] = RES_ALPHA * dot + (1.0 - RES_ALPHA) * mlp


def _mlp_score(u, it, ub, ib, om, gam, bet, w1, b1, w2, b2):
    return pl.pallas_call(
        _mlp_body,
        out_shape=jax.ShapeDtypeStruct((BATCH, 1), jnp.float32),
    )(u, it, ub, ib, om, gam, bet, w1, b1, w2, b2)


def kernel(users, items, edge_index, edge_weight, user_emb, item_emb,
           user_bias_w, item_bias_w, ln_gamma, ln_beta, W1, b1, W2, b2):
    x0 = jnp.concatenate([user_emb, item_emb], axis=0)
    zpad_i = jnp.zeros((NW, PAD), jnp.int32)
    srcp = jnp.concatenate(
        [edge_index[0].reshape(NW, EPW), zpad_i], axis=1).reshape(NW, NCHUNK, EC)
    dstp = jnp.concatenate(
        [edge_index[1].reshape(NW, EPW), zpad_i], axis=1).reshape(NW, NCHUNK, EC)
    zpad_f = jnp.zeros((NW, PAD), jnp.float32)
    ew = jnp.concatenate(
        [edge_weight.reshape(NW, EPW), zpad_f], axis=1).reshape(NW, NCHUNK, EC)
    e3 = jnp.stack([srcp, dstp], axis=2)

    x = x0
    ssum = x0
    out_mean = None
    for layer in range(N_LAYERS):
        p = _propagate_kernel()(x, e3, ew)
        if layer < N_LAYERS - 1:
            x, ssum = _combine(p, ssum)
        else:
            out_mean = _combine_last(p, ssum)

    gathered = _batch_gather_kernel()(out_mean, user_bias_w, item_bias_w,
                                      users, items)
    u, it, ub, ib = gathered[0], gathered[1], gathered[2], gathered[3]
    score = _mlp_score(u, it, ub, ib, out_mean,
                       ln_gamma.reshape(1, IN_DIM), ln_beta.reshape(1, IN_DIM),
                       W1, b1.reshape(1, HIDDEN), W2, b2.reshape(1, 1))
    return score[:, 0]


# speedup vs baseline: 2.3642x; 2.3642x over previous
import functools

import jax
import jax.numpy as jnp
from jax import lax
from jax.experimental import pallas as pl
from jax.experimental.pallas import tpu as pltpu
from jax.experimental.pallas import tpu_sc as plsc

NUM_USERS = 5000
NUM_ITEMS = 5000
N_NODES = NUM_USERS + NUM_ITEMS
N_EDGES = 320000
D = 128
N_LAYERS = 4
BATCH = 4096
BIAS_SCALE = 0.5
RES_ALPHA = 0.3
EPS = 1e-5
IN_DIM = 5 * D
HIDDEN = 128

NC = 2
NS = 16
NW = NC * NS
LANES = 16

EPW = N_EDGES // NW
EC = 64
NCHUNK = 157
EPW_PAD = NCHUNK * EC
PAD = EPW_PAD - EPW
RPT = 624
TAIL = N_NODES - RPT * NS
ZCH = RPT // EC
ZREM = RPT - ZCH * EC

B_PER_W = BATCH // NW

@functools.cache
def _get_mesh():
    return plsc.VectorSubcoreMesh(
        core_axis_name="c", subcore_axis_name="s",
        num_cores=NC, num_subcores=NS)


@functools.cache
def _propagate_kernel():
    return pl.kernel(
        _propagate_body,
        out_type=jax.ShapeDtypeStruct((NC, N_NODES, D), jnp.float32),
        mesh=_get_mesh(),
        scratch_types=[
            pltpu.VMEM((4, 2, EC), jnp.int32),
            pltpu.VMEM((4, EC), jnp.float32),
            pltpu.VMEM((4, EC), jnp.int32),
            pltpu.VMEM((EC, D), jnp.float32),
            pltpu.VMEM((EC, D), jnp.float32),
            pltpu.VMEM((EC, D), jnp.float32),
            pltpu.VMEM((EC, D), jnp.float32),
            pltpu.VMEM_SHARED((N_NODES, D), jnp.float32),
            [pltpu.SemaphoreType.DMA] * 4,
            [pltpu.SemaphoreType.DMA] * 4,
            [pltpu.SemaphoreType.DMA] * 4,
        ],
    )


def _propagate_body(x_hbm, e_hbm, ew_hbm, out_hbm,
                    ebuf_v, ewbuf_v, dstc_v, rows0_v, rows1_v, rows2_v,
                    rows3_v, acc_sh, gsem, esem, ssem):
    c = lax.axis_index("c")
    s = lax.axis_index("s")
    wid = c * NS + s

    zero16 = jnp.zeros((LANES,), jnp.float32)
    for r in range(EC):
        for j in range(D // LANES):
            rows0_v[r, pl.ds(j * LANES, LANES)] = zero16
    for k in range(ZCH):
        acc_off = s * RPT + k * EC
        pltpu.sync_copy(rows0_v, acc_sh.at[pl.ds(acc_off, EC)])
    pltpu.sync_copy(rows0_v.at[pl.ds(0, ZREM)],
                    acc_sh.at[pl.ds(s * RPT + ZCH * EC, ZREM)])

    @pl.when(s == NS - 1)
    def _zero_tail():
        pltpu.sync_copy(rows0_v.at[pl.ds(0, TAIL)],
                        acc_sh.at[pl.ds(RPT * NS, TAIL)])

    plsc.subcore_barrier()

    rows = (rows0_v, rows1_v, rows2_v, rows3_v)

    def scale(p):
        buf = rows[p]
        for j in range(EC // LANES):
            sl = pl.ds(j * LANES, LANES)
            dstc_v[p, sl] = ebuf_v[p, 1, sl]
        for g in range(EC // LANES):
            wvec = ewbuf_v[p, pl.ds(g * LANES, LANES)]
            for l in range(LANES):
                i = g * LANES + l
                wi = wvec[l]
                for j in range(D // LANES):
                    sl = pl.ds(j * LANES, LANES)
                    buf[i, sl] = buf[i, sl] * wi

    def issue_scatter(p):
        pltpu.async_copy(rows[p], acc_sh.at[dstc_v.at[p]], ssem[p],
                         add=True)

    def wait_scatter(p):
        pltpu.make_async_copy(rows[p], acc_sh.at[dstc_v.at[p]],
                              ssem[p]).wait()

    def issue_edges(k, p):
        pltpu.async_copy(e_hbm.at[wid, k], ebuf_v.at[p], esem[p])
        pltpu.async_copy(ew_hbm.at[wid, k], ewbuf_v.at[p], esem[p])

    def wait_edges(p):
        pltpu.make_async_copy(e_hbm.at[wid, 0], ebuf_v.at[p],
                              esem[p]).wait()
        pltpu.make_async_copy(ew_hbm.at[wid, 0], ewbuf_v.at[p],
                              esem[p]).wait()

    def issue_gather(p):
        pltpu.async_copy(x_hbm.at[ebuf_v.at[p, 0]], rows[p], gsem[p])

    def wait_gather(p):
        pltpu.make_async_copy(x_hbm.at[ebuf_v.at[p, 0]], rows[p],
                              gsem[p]).wait()

    pltpu.sync_copy(e_hbm.at[wid, 0], ebuf_v.at[0])
    pltpu.sync_copy(ew_hbm.at[wid, 0], ewbuf_v.at[0])
    issue_gather(0)
    issue_edges(1, 1)
    issue_edges(2, 2)

    issue_edges(3, 3)
    wait_edges(1)
    issue_gather(1)
    wait_edges(2)
    issue_gather(2)
    wait_gather(0)
    scale(0)
    issue_scatter(0)

    issue_edges(4, 0)
    wait_edges(3)
    issue_gather(3)
    wait_gather(1)
    scale(1)
    issue_scatter(1)

    def steady(k, p):
        p2 = (p + 2) % 4
        issue_edges(k + 3, (p + 3) % 4)
        wait_scatter(p2)
        wait_edges(p2)
        issue_gather(p2)
        wait_gather(p)
        scale(p)
        issue_scatter(p)

    def body(t, carry):
        k = 4 * t + 2
        steady(k, 2)
        steady(k + 1, 3)
        steady(k + 2, 0)
        steady(k + 3, 1)
        return carry

    lax.fori_loop(0, (NCHUNK - 5) // 4, body, 0)

    wait_scatter(0)
    wait_edges(0)
    issue_gather(0)
    wait_gather(2)
    scale(2)
    issue_scatter(2)

    wait_scatter(1)
    wait_gather(3)
    scale(3)
    issue_scatter(3)

    wait_scatter(2)
    wait_gather(0)
    scale(0)
    issue_scatter(0)

    wait_scatter(3)
    wait_scatter(0)

    plsc.subcore_barrier()
    for k in range(ZCH):
        r0 = s * RPT + k * EC
        pltpu.sync_copy(acc_sh.at[pl.ds(r0, EC)], rows0_v)
        pltpu.sync_copy(rows0_v, out_hbm.at[c, pl.ds(r0, EC)])
    rrem = s * RPT + ZCH * EC
    pltpu.sync_copy(acc_sh.at[pl.ds(rrem, ZREM)], rows0_v.at[pl.ds(0, ZREM)])
    pltpu.sync_copy(rows0_v.at[pl.ds(0, ZREM)],
                    out_hbm.at[c, pl.ds(rrem, ZREM)])

    @pl.when(s == NS - 1)
    def _write_tail():
        pltpu.sync_copy(acc_sh.at[pl.ds(RPT * NS, TAIL)],
                        rows1_v.at[pl.ds(0, TAIL)])
        pltpu.sync_copy(rows1_v.at[pl.ds(0, TAIL)],
                        out_hbm.at[c, pl.ds(RPT * NS, TAIL)])


@functools.cache
def _batch_gather_kernel():
    return pl.kernel(
        _batch_gather_body,
        out_type=jax.ShapeDtypeStruct((4, BATCH, D), jnp.float32),
        mesh=_get_mesh(),
        scratch_types=[
            pltpu.VMEM((B_PER_W,), jnp.int32),
            pltpu.VMEM((B_PER_W,), jnp.int32),
            pltpu.VMEM((B_PER_W,), jnp.int32),
            pltpu.VMEM((B_PER_W, D), jnp.float32),
            pltpu.VMEM((B_PER_W, D), jnp.float32),
            pltpu.VMEM((B_PER_W, D), jnp.float32),
            pltpu.VMEM((B_PER_W, D), jnp.float32),
            pltpu.SemaphoreType.DMA,
        ],
    )


def _batch_gather_body(om_hbm, ubw_hbm, ibw_hbm, users_hbm, items_hbm,
                       out_hbm,
                       uidx_v, iidx_v, iidx2_v, bu_v, bi_v, bub_v, bib_v, sem):
    c = lax.axis_index("c")
    s = lax.axis_index("s")
    wid = c * NS + s
    base = pl.multiple_of(wid * B_PER_W, 8)

    pltpu.sync_copy(users_hbm.at[pl.ds(base, B_PER_W)], uidx_v)
    pltpu.sync_copy(items_hbm.at[pl.ds(base, B_PER_W)], iidx_v)
    off16 = jnp.full((LANES,), NUM_USERS, jnp.int32)
    for j in range(B_PER_W // LANES):
        sl = pl.ds(j * LANES, LANES)
        iidx2_v[sl] = iidx_v[sl] + off16

    cu = pltpu.async_copy(om_hbm.at[uidx_v], bu_v, sem)
    ci = pltpu.async_copy(om_hbm.at[iidx2_v], bi_v, sem)
    cub = pltpu.async_copy(ubw_hbm.at[uidx_v], bub_v, sem)
    cib = pltpu.async_copy(ibw_hbm.at[iidx_v], bib_v, sem)
    cu.wait()
    ci.wait()
    cub.wait()
    cib.wait()
    pltpu.sync_copy(bu_v, out_hbm.at[0, pl.ds(base, B_PER_W)])
    pltpu.sync_copy(bi_v, out_hbm.at[1, pl.ds(base, B_PER_W)])
    pltpu.sync_copy(bub_v, out_hbm.at[2, pl.ds(base, B_PER_W)])
    pltpu.sync_copy(bib_v, out_hbm.at[3, pl.ds(base, B_PER_W)])


_CROWS = 1000


def _combine_body(p_ref, s_ref, x_ref, sum_ref):
    x = p_ref[0] + p_ref[1]
    x_ref[...] = x
    sum_ref[...] = s_ref[...] + x


def _combine(p, ssum):
    return pl.pallas_call(
        _combine_body,
        grid=(N_NODES // _CROWS,),
        in_specs=[
            pl.BlockSpec((NC, _CROWS, D), lambda i: (0, i, 0)),
            pl.BlockSpec((_CROWS, D), lambda i: (i, 0)),
        ],
        out_specs=[
            pl.BlockSpec((_CROWS, D), lambda i: (i, 0)),
            pl.BlockSpec((_CROWS, D), lambda i: (i, 0)),
        ],
        out_shape=[
            jax.ShapeDtypeStruct((N_NODES, D), jnp.float32),
            jax.ShapeDtypeStruct((N_NODES, D), jnp.float32),
        ],
    )(p, ssum)


def _combine_last_body(p_ref, s_ref, out_ref):
    out_ref[...] = (s_ref[...] + p_ref[0] + p_ref[1]) * (1.0 / (N_LAYERS + 1))


def _combine_last(p, ssum):
    return pl.pallas_call(
        _combine_last_body,
        grid=(N_NODES // _CROWS,),
        in_specs=[
            pl.BlockSpec((NC, _CROWS, D), lambda i: (0, i, 0)),
            pl.BlockSpec((_CROWS, D), lambda i: (i, 0)),
        ],
        out_specs=pl.BlockSpec((_CROWS, D), lambda i: (i, 0)),
        out_shape=jax.ShapeDtypeStruct((N_NODES, D), jnp.float32),
    )(p, ssum)


def _mlp_body(u_ref, i_ref, ub_ref, ib_ref, om_ref, gam_ref, bet_ref,
              w1_ref, b1_ref, w2_ref, b2_ref, out_ref):
    u = u_ref[...]
    it = i_ref[...]
    ub = ub_ref[...] * BIAS_SCALE
    ib = ib_ref[...] * BIAS_SCALE
    om = om_ref[...]
    g_row = (jnp.sum(om[:NUM_USERS], axis=0, keepdims=True)
             + jnp.sum(om[NUM_USERS:], axis=0, keepdims=True)) * (1.0 / NUM_USERS)

    pieces = [u, it, ub, ib]
    tot = jnp.zeros((BATCH, 1), jnp.float32)
    for p in pieces:
        tot = tot + jnp.sum(p, axis=1, keepdims=True)
    tot = tot + jnp.sum(g_row)
    mu = tot * (1.0 / IN_DIM)

    ssq = jnp.zeros((BATCH, 1), jnp.float32)
    for p in pieces:
        d = p - mu
        ssq = ssq + jnp.sum(d * d, axis=1, keepdims=True)
    dg = g_row - mu
    ssq = ssq + jnp.sum(dg * dg, axis=1, keepdims=True)
    inv = lax.rsqrt(ssq * (1.0 / IN_DIM) + EPS)

    h = jnp.zeros((BATCH, HIDDEN), jnp.float32)
    for k in range(4):
        d = (pieces[k] - mu) * inv
        xn = d * gam_ref[:, k * D:(k + 1) * D] + bet_ref[:, k * D:(k + 1) * D]
        h = h + jnp.dot(xn, w1_ref[k * D:(k + 1) * D, :],
                        preferred_element_type=jnp.float32)
    dgn = dg * inv
    xng = dgn * gam_ref[:, 4 * D:] + bet_ref[:, 4 * D:]
    h = h + jnp.dot(xng, w1_ref[4 * D:, :], preferred_element_type=jnp.float32)
    h = jnp.maximum(h + b1_ref[...], 0.0)

    mlp = jnp.dot(h, w2_ref[...], preferred_element_type=jnp.float32) + b2_ref[...]
    dot = jnp.sum(u * it, axis=1, keepdims=True)
    out_ref[...] = RES_ALPHA * dot + (1.0 - RES_ALPHA) * mlp


def _mlp_score(u, it, ub, ib, om, gam, bet, w1, b1, w2, b2):
    return pl.pallas_call(
        _mlp_body,
        out_shape=jax.ShapeDtypeStruct((BATCH, 1), jnp.float32),
    )(u, it, ub, ib, om, gam, bet, w1, b1, w2, b2)


def kernel(users, items, edge_index, edge_weight, user_emb, item_emb,
           user_bias_w, item_bias_w, ln_gamma, ln_beta, W1, b1, W2, b2):
    x0 = jnp.concatenate([user_emb, item_emb], axis=0)
    zpad_i = jnp.zeros((NW, PAD), jnp.int32)
    srcp = jnp.concatenate(
        [edge_index[0].reshape(NW, EPW), zpad_i], axis=1).reshape(NW, NCHUNK, EC)
    dstp = jnp.concatenate(
        [edge_index[1].reshape(NW, EPW), zpad_i], axis=1).reshape(NW, NCHUNK, EC)
    zpad_f = jnp.zeros((NW, PAD), jnp.float32)
    ew = jnp.concatenate(
        [edge_weight.reshape(NW, EPW), zpad_f], axis=1).reshape(NW, NCHUNK, EC)
    e3 = jnp.stack([srcp, dstp], axis=2)

    x = x0
    ssum = x0
    out_mean = None
    for layer in range(N_LAYERS):
        p = _propagate_kernel()(x, e3, ew)
        if layer < N_LAYERS - 1:
            x, ssum = _combine(p, ssum)
        else:
            out_mean = _combine_last(p, ssum)

    gathered = _batch_gather_kernel()(out_mean, user_bias_w, item_bias_w,
                                      users, items)
    u, it, ub, ib = gathered[0], gathered[1], gathered[2], gathered[3]
    score = _mlp_score(u, it, ub, ib, out_mean,
                       ln_gamma.reshape(1, IN_DIM), ln_beta.reshape(1, IN_DIM),
                       W1, b1.reshape(1, HIDDEN), W2, b2.reshape(1, 1))
    return score[:, 0]
